# Initial kernel scaffold; baseline (speedup 1.0000x reference)
import jax
import jax.numpy as jnp
from jax.experimental import pallas as pl


def kernel(x):
    raise NotImplementedError("write your pallas kernel here")


# speedup vs baseline: 22.0259x; 22.0259x over previous
import math

import jax
import jax.numpy as jnp
from jax.experimental import pallas as pl
from jax.experimental.pallas import tpu as pltpu

_ORI = 9
_CELL = 8
_BLK = 3
_EPS = 1e-5
_H = 512
_W = 512
_NC = _H // _CELL
_NB = _NC - _BLK + 1
_K = _BLK * _BLK * _ORI


def _hog_body(x_ref, out_ref):
    img = jnp.sqrt(x_ref[0])
    f32 = jnp.float32

    rows = jax.lax.broadcasted_iota(jnp.int32, (_H, _W), 0)
    cols = jax.lax.broadcasted_iota(jnp.int32, (_H, _W), 1)
    zrow = jnp.zeros((1, _W), f32)
    zcol = jnp.zeros((_H, 1), f32)
    up = jnp.concatenate([img[1:, :], zrow], axis=0)
    dn = jnp.concatenate([zrow, img[:-1, :]], axis=0)
    g_row = jnp.where((rows == 0) | (rows == _H - 1), 0.0, up - dn)
    lf = jnp.concatenate([img[:, 1:], zcol], axis=1)
    rt = jnp.concatenate([zcol, img[:, :-1]], axis=1)
    g_col = jnp.where((cols == 0) | (cols == _W - 1), 0.0, lf - rt)

    mag = jnp.sqrt(g_row * g_row + g_col * g_col)

    neg = g_row < 0.0
    grnz = g_row != 0.0
    a = jnp.abs(g_row)
    b = jnp.where(neg, -g_col, g_col)

    jj = jax.lax.broadcasted_iota(jnp.int32, (_W, _NC), 0)
    cc = jax.lax.broadcasted_iota(jnp.int32, (_W, _NC), 1)
    pt = jnp.where((jj // _CELL) == cc, 1.0, 0.0).astype(f32)

    def cell_pool(field):
        r = jnp.sum(field.reshape(_NC, _CELL, _W), axis=1)
        return jax.lax.dot(r, pt, precision=jax.lax.Precision.HIGHEST,
                           preferred_element_type=f32)

    pooled = [cell_pool(mag)]
    for k in range(1, _ORI):
        th = math.radians(20.0 * k)
        d = a * f32(math.cos(th)) - b * f32(math.sin(th))
        m_k = jnp.where((d >= 0.0) & grnz, mag, 0.0)
        pooled.append(cell_pool(m_k))

    inv_area = f32(1.0 / (_CELL * _CELL))
    hist = []
    for o in range(_ORI):
        hi = pooled[o] - pooled[o + 1] if o + 1 < _ORI else pooled[o]
        hist.append(hi * inv_area)

    ssq = hist[0] * hist[0]
    for o in range(1, _ORI):
        ssq = ssq + hist[o] * hist[o]
    bs = jnp.zeros((_NB, _NB), f32)
    for i in range(_BLK):
        for j in range(_BLK):
            bs = bs + ssq[i:i + _NB, j:j + _NB]
    ninv = jax.lax.rsqrt(bs + f32(_EPS * _EPS))

    for i in range(_BLK):
        for j in range(_BLK):
            for o in range(_ORI):
                k = (i * _BLK + j) * _ORI + o
                out_ref[0, k] = hist[o][i:i + _NB, j:j + _NB] * ninv


def kernel(x):
    B = x.shape[0]
    xs = x.reshape(B, _H, _W)
    out = pl.pallas_call(
        _hog_body,
        grid=(B,),
        in_specs=[pl.BlockSpec((1, _H, _W), lambda b: (b, 0, 0))],
        out_specs=pl.BlockSpec((1, _K, _NB, _NB), lambda b: (b, 0, 0, 0)),
        out_shape=jax.ShapeDtypeStruct((B, _K, _NB, _NB), jnp.float32),
        compiler_params=pltpu.CompilerParams(
            dimension_semantics=("arbitrary",),
        ),
    )(xs)
    out = out.reshape(B, _BLK, _BLK, _ORI, _NB, _NB)
    out = out.transpose(0, 4, 5, 1, 2, 3)
    return out.reshape(B, -1)


# speedup vs baseline: 22.7956x; 1.0349x over previous
import math

import jax
import jax.numpy as jnp
from jax.experimental import pallas as pl
from jax.experimental.pallas import tpu as pltpu

_ORI = 9
_CELL = 8
_BLK = 3
_EPS = 1e-5
_H = 512
_W = 512
_NC = _H // _CELL
_NB = _NC - _BLK + 1
_K = _BLK * _BLK * _ORI


def _hog_body(x_ref, out_ref):
    img = jnp.sqrt(x_ref[0])
    f32 = jnp.float32

    rows = jax.lax.broadcasted_iota(jnp.int32, (_H, _W), 0)
    cols = jax.lax.broadcasted_iota(jnp.int32, (_H, _W), 1)
    zrow = jnp.zeros((1, _W), f32)
    zcol = jnp.zeros((_H, 1), f32)
    up = jnp.concatenate([img[1:, :], zrow], axis=0)
    dn = jnp.concatenate([zrow, img[:-1, :]], axis=0)
    g_row = jnp.where((rows == 0) | (rows == _H - 1), 0.0, up - dn)
    lf = jnp.concatenate([img[:, 1:], zcol], axis=1)
    rt = jnp.concatenate([zcol, img[:, :-1]], axis=1)
    g_col = jnp.where((cols == 0) | (cols == _W - 1), 0.0, lf - rt)

    mag = jnp.sqrt(g_row * g_row + g_col * g_col)
    mag_bf = mag.astype(jnp.bfloat16)

    a = jnp.abs(g_row)
    b = jnp.where(g_row < 0.0, -g_col, g_col)
    t = jnp.where(a == 0.0, jnp.inf, b * (1.0 / a))

    jj = jax.lax.broadcasted_iota(jnp.int32, (_W, _NC), 0)
    cc = jax.lax.broadcasted_iota(jnp.int32, (_W, _NC), 1)
    pt = jnp.where((jj // _CELL) == cc, 1.0, 0.0).astype(jnp.bfloat16)

    def cell_pool(field_bf):
        z = jax.lax.dot(field_bf, pt, preferred_element_type=f32)
        return jnp.sum(z.reshape(_NC, _CELL, _NC), axis=1)

    zero_bf = jnp.zeros((), jnp.bfloat16)
    ind = [None] + [t <= f32(1.0 / math.tan(math.radians(20.0 * k)))
                    for k in range(1, _ORI)]
    inv_area = f32(1.0 / (_CELL * _CELL))
    hist = []
    for o in range(_ORI):
        if o == 0:
            in_bin = ~ind[1]
        elif o == _ORI - 1:
            in_bin = ind[_ORI - 1]
        else:
            in_bin = ind[o] & ~ind[o + 1]
        m_o = jnp.where(in_bin, mag_bf, zero_bf)
        hist.append(cell_pool(m_o) * inv_area)

    ssq = hist[0] * hist[0]
    for o in range(1, _ORI):
        ssq = ssq + hist[o] * hist[o]
    bs = jnp.zeros((_NB, _NB), f32)
    for i in range(_BLK):
        for j in range(_BLK):
            bs = bs + ssq[i:i + _NB, j:j + _NB]
    ninv = jax.lax.rsqrt(bs + f32(_EPS * _EPS))

    for i in range(_BLK):
        for j in range(_BLK):
            for o in range(_ORI):
                k = (i * _BLK + j) * _ORI + o
                out_ref[0, k] = hist[o][i:i + _NB, j:j + _NB] * ninv


def kernel(x):
    B = x.shape[0]
    xs = x.reshape(B, _H, _W)
    out = pl.pallas_call(
        _hog_body,
        grid=(B,),
        in_specs=[pl.BlockSpec((1, _H, _W), lambda b: (b, 0, 0))],
        out_specs=pl.BlockSpec((1, _K, _NB, _NB), lambda b: (b, 0, 0, 0)),
        out_shape=jax.ShapeDtypeStruct((B, _K, _NB, _NB), jnp.float32),
        compiler_params=pltpu.CompilerParams(
            dimension_semantics=("arbitrary",),
        ),
    )(xs)
    out = out.reshape(B, _BLK, _BLK, _ORI, _NB, _NB)
    out = out.transpose(0, 4, 5, 1, 2, 3)
    return out.reshape(B, -1)
